# baseline (speedup 1.0000x reference)
import functools

import jax
import jax.numpy as jnp
import numpy as np
from jax import lax
from jax.experimental import pallas as pl
from jax.experimental.pallas import tpu as pltpu
from jax.experimental.pallas import tpu_sc as plsc

ROWS = 128
COLS = 32768
K = 64
LANES = 16
NV = COLS // LANES
HIST = 256
CAP = 8192
CAPB = CAP + LANES
NSEG = 4
QCAP = CAP // NSEG
NC = 2
NS = 16
NW = NC * NS
ROWS_PER_W = ROWS // NW

ABS_MASK = np.int32(0x7FFFFFFF)


def _body(in_hbm, out_hbm, x0, x1, hist, av, ai, bv, bi, gsum_s,
          si0, si1, so0, so1):
    wid = lax.axis_index("s") * NC + lax.axis_index("c")
    iota = lax.iota(jnp.int32, LANES)
    ones16 = jnp.ones((LANES,), jnp.int32)
    zeros16 = jnp.zeros((LANES,), jnp.int32)

    def clear_hist():
        @pl.loop(0, HIST // LANES)
        def _(h):
            hist[pl.ds(h * LANES, LANES)] = zeros16

    def hist_at(b):
        return hist[pl.ds(b, LANES)][0]

    def scan_hist(needed):
        for g in range(HIST // LANES):
            gsum_s[g] = jnp.sum(hist[pl.ds(g * LANES, LANES)])

        def gcond(st):
            g, cum = st
            return (cum < needed) & (g > 0)

        def gstep(st):
            g, cum = st
            g2 = g - 1
            return g2, cum + gsum_s[g2]

        gstar, gcum = lax.while_loop(
            gcond, gstep, (np.int32(HIST // LANES), np.int32(0)))
        base = gstar * LANES

        def cond(st):
            b, cum = st
            return (cum < needed) & (b > base)

        def step(st):
            b, cum = st
            b2 = b - 1
            return b2, cum + hist_at(b2)

        bstar, cum = lax.while_loop(
            cond, step, (base + LANES, gcum - gsum_s[gstar]))
        return bstar, needed - (cum - hist_at(bstar))

    def do_row(row, xrow):
        @pl.loop(0, NV // NSEG, init_carry=(zeros16,) * NSEG, unroll=2)
        def mxs(v, mxs):
            out = []
            for q in range(NSEG):
                vq = q * (NV // NSEG) + v
                x = xrow[pl.ds(vq * LANES, LANES)]
                key = plsc.bitcast(x, jnp.int32) & ABS_MASK
                out.append(jnp.maximum(mxs[q], key))
            return tuple(out)

        gmin = mxs[0]
        for q in range(1, NSEG):
            gmin = jnp.minimum(gmin, mxs[q])
        tkey = jnp.min(gmin)

        zrow = jnp.zeros((LANES,), jnp.float32)

        @pl.loop(0, NV // NSEG, init_carry=(np.int32(0),) * NSEG, unroll=2)
        def segn(v, offs):
            new_offs = []
            for q in range(NSEG):
                vq = q * (NV // NSEG) + v
                x = xrow[pl.ds(vq * LANES, LANES)]
                raw = plsc.bitcast(x, jnp.int32)
                key = raw & ABS_MASK
                meq = key >= tkey
                off = offs[q]
                base = q * QCAP
                pc = plsc.all_reduce_population_count(meq)[0]

                @pl.when(pc > 0)
                def _():
                    plsc.store_compressed(
                        av.at[pl.ds(base + off, LANES)], raw, mask=meq)
                    plsc.store_compressed(
                        ai.at[pl.ds(base + off, LANES)], vq * LANES + iota,
                        mask=meq)

                xrow[pl.ds(vq * LANES, LANES)] = zrow
                new_offs.append(jnp.minimum(off + pc, QCAP - LANES))
            return tuple(new_offs)

        def refine(sv, si, dv, di, n, needed, shift, nbits):
            bmask = np.int32((1 << nbits) - 1)
            trips = (n + LANES - 1) // LANES
            clear_hist()

            @pl.loop(0, trips)
            def _(v):
                m = (v * LANES + iota) < n
                key = sv[pl.ds(v * LANES, LANES)] & ABS_MASK
                plsc.addupdate_scatter(
                    hist, [(key >> shift) & bmask], ones16, mask=m)

            bstar, needed2 = scan_hist(needed)

            @pl.loop(0, trips, init_carry=np.int32(0))
            def n2(v, off):
                lanem = (v * LANES + iota) < n
                raw = sv[pl.ds(v * LANES, LANES)]
                idx = si[pl.ds(v * LANES, LANES)]
                b = ((raw & ABS_MASK) >> shift) & bmask
                mgt = lanem & (b > bstar)
                plsc.store_scatter(
                    xrow, [idx], plsc.bitcast(raw, jnp.float32), mask=mgt)
                meq = lanem & (b == bstar)
                plsc.store_compressed(dv.at[pl.ds(off, LANES)], raw, mask=meq)
                plsc.store_compressed(di.at[pl.ds(off, LANES)], idx, mask=meq)
                pc = plsc.all_reduce_population_count(meq)[0]
                return jnp.minimum(off + pc, CAP)

            return n2, needed2

        def refine_seg(sv, si, dv, di, ns, needed, shift, nbits):
            bmask = np.int32((1 << nbits) - 1)
            clear_hist()
            for q in range(NSEG):
                trips = (ns[q] + LANES - 1) // LANES

                @pl.loop(0, trips)
                def _(v):
                    m = (v * LANES + iota) < ns[q]
                    key = sv[pl.ds(q * QCAP + v * LANES, LANES)] & ABS_MASK
                    plsc.addupdate_scatter(
                        hist, [(key >> shift) & bmask], ones16, mask=m)

            bstar, needed2 = scan_hist(needed)

            n2 = np.int32(0)
            for q in range(NSEG):
                trips = (ns[q] + LANES - 1) // LANES

                @pl.loop(0, trips, init_carry=n2)
                def n2(v, off):
                    lanem = (v * LANES + iota) < ns[q]
                    raw = sv[pl.ds(q * QCAP + v * LANES, LANES)]
                    idx = si[pl.ds(q * QCAP + v * LANES, LANES)]
                    b = ((raw & ABS_MASK) >> shift) & bmask
                    mgt = lanem & (b > bstar)
                    plsc.store_scatter(
                        xrow, [idx], plsc.bitcast(raw, jnp.float32), mask=mgt)
                    meq = lanem & (b == bstar)
                    plsc.store_compressed(
                        dv.at[pl.ds(off, LANES)], raw, mask=meq)
                    plsc.store_compressed(
                        di.at[pl.ds(off, LANES)], idx, mask=meq)
                    pc = plsc.all_reduce_population_count(meq)[0]
                    return jnp.minimum(off + pc, CAP)

            return n2, needed2

        needed = np.int32(K)
        nB, needed = refine_seg(av, ai, bv, bi, segn, needed, 23, 8)
        nA2, needed = refine(bv, bi, av, ai, nB, needed, 15, 8)
        nB2, needed = refine(av, ai, bv, bi, nA2, needed, 7, 8)
        nT, needed = refine(bv, bi, av, ai, nB2, needed, 0, 7)

        tie_trips = (jnp.minimum(nT, needed) + LANES - 1) // LANES

        @pl.loop(0, tie_trips)
        def _(v):
            posv = v * LANES + iota
            m = (posv < nT) & (posv < needed)
            raw = av[pl.ds(v * LANES, LANES)]
            idx = ai[pl.ds(v * LANES, LANES)]
            plsc.store_scatter(
                xrow, [idx], plsc.bitcast(raw, jnp.float32), mask=m)

    xbufs = (x0, x1)
    sin = (si0, si1)
    sout = (so0, so1)
    row0 = wid * ROWS_PER_W
    in_d = [pltpu.async_copy(in_hbm.at[row0], x0, si0), None]
    out_d = [None, None]
    for r in range(ROWS_PER_W):
        b = r % 2
        in_d[b].wait()
        if r + 1 < ROWS_PER_W:
            nb = (r + 1) % 2
            if out_d[nb] is not None:
                out_d[nb].wait()
            in_d[nb] = pltpu.async_copy(
                in_hbm.at[row0 + r + 1], xbufs[nb], sin[nb])
        do_row(row0 + r, xbufs[b])
        out_d[b] = pltpu.async_copy(xbufs[b], out_hbm.at[row0 + r], sout[b])
    out_d[(ROWS_PER_W - 2) % 2].wait()
    out_d[(ROWS_PER_W - 1) % 2].wait()


@jax.jit
def kernel(input_):
    mesh = plsc.VectorSubcoreMesh(
        core_axis_name="c", subcore_axis_name="s",
        num_cores=NC, num_subcores=NS)
    f = pl.kernel(
        _body,
        out_type=jax.ShapeDtypeStruct((ROWS, COLS), jnp.float32),
        mesh=mesh,
        scratch_types=[
            pltpu.VMEM((COLS,), jnp.float32),
            pltpu.VMEM((COLS,), jnp.float32),
            pltpu.VMEM((HIST + LANES,), jnp.int32),
            pltpu.VMEM((CAPB,), jnp.int32),
            pltpu.VMEM((CAPB,), jnp.int32),
            pltpu.VMEM((CAPB,), jnp.int32),
            pltpu.VMEM((CAPB,), jnp.int32),
            pltpu.SMEM((LANES,), jnp.int32),
            pltpu.SemaphoreType.DMA,
            pltpu.SemaphoreType.DMA,
            pltpu.SemaphoreType.DMA,
            pltpu.SemaphoreType.DMA,
        ],
        compiler_params=pltpu.CompilerParams(needs_layout_passes=False),
        name="topk_abs_mask_sc",
    )
    return f(input_)


# speedup vs baseline: 1.6353x; 1.6353x over previous
import functools

import jax
import jax.numpy as jnp
import numpy as np
from jax import lax
from jax.experimental import pallas as pl
from jax.experimental.pallas import tpu as pltpu
from jax.experimental.pallas import tpu_sc as plsc

ROWS = 128
COLS = 32768
K = 64
LANES = 16
NV = COLS // LANES
HIST = 256
CAP = 8192
CAPB = CAP + LANES
NSEG = 4
QCAP = CAP // NSEG
NC = 2
NS = 16
NW = NC * NS
ROWS_PER_W = ROWS // NW

ABS_MASK = np.int32(0x7FFFFFFF)


def _body(in_hbm, out_hbm, x0, x1, hist, av, ai, bv, bi, gsum_s,
          si0, si1, so0, so1):
    wid = lax.axis_index("s") * NC + lax.axis_index("c")
    iota = lax.iota(jnp.int32, LANES)
    ones16 = jnp.ones((LANES,), jnp.int32)
    zeros16 = jnp.zeros((LANES,), jnp.int32)

    def clear_hist():
        @pl.loop(0, HIST // LANES)
        def _(h):
            hist[pl.ds(h * LANES, LANES)] = zeros16

    def hist_at(b):
        return hist[pl.ds(b, LANES)][0]

    def scan_hist(needed):
        for g in range(HIST // LANES):
            gsum_s[g] = jnp.sum(hist[pl.ds(g * LANES, LANES)])

        def gcond(st):
            g, cum = st
            return (cum < needed) & (g > 0)

        def gstep(st):
            g, cum = st
            g2 = g - 1
            return g2, cum + gsum_s[g2]

        gstar, gcum = lax.while_loop(
            gcond, gstep, (np.int32(HIST // LANES), np.int32(0)))
        base = gstar * LANES

        def cond(st):
            b, cum = st
            return (cum < needed) & (b > base)

        def step(st):
            b, cum = st
            b2 = b - 1
            return b2, cum + hist_at(b2)

        bstar, cum = lax.while_loop(
            cond, step, (base + LANES, gcum - gsum_s[gstar]))
        return bstar, needed - (cum - hist_at(bstar))

    def do_row(row, xrow):
        @pl.loop(0, NV // NSEG, init_carry=(zeros16,) * NSEG, unroll=2)
        def mxs(v, mxs):
            out = []
            for q in range(NSEG):
                vq = q * (NV // NSEG) + v
                x = xrow[pl.ds(vq * LANES, LANES)]
                key = plsc.bitcast(x, jnp.int32) & ABS_MASK
                out.append(jnp.maximum(mxs[q], key))
            return tuple(out)

        gmin = mxs[0]
        for q in range(1, NSEG):
            gmin = jnp.minimum(gmin, mxs[q])
        tkey = jnp.min(gmin)

        zrow = jnp.zeros((LANES,), jnp.float32)

        @pl.loop(0, NV // NSEG, init_carry=(np.int32(0),) * NSEG, unroll=4)
        def segn(v, offs):
            new_offs = []
            for q in range(NSEG):
                vq = q * (NV // NSEG) + v
                x = xrow[pl.ds(vq * LANES, LANES)]
                raw = plsc.bitcast(x, jnp.int32)
                key = raw & ABS_MASK
                meq = key >= tkey
                off = offs[q]
                base = q * QCAP
                plsc.store_compressed(
                    av.at[pl.ds(base + off, LANES)], raw, mask=meq)
                plsc.store_compressed(
                    ai.at[pl.ds(base + off, LANES)], vq * LANES + iota,
                    mask=meq)
                xrow[pl.ds(vq * LANES, LANES)] = zrow
                pc = plsc.all_reduce_population_count(meq)[0]
                new_offs.append(jnp.minimum(off + pc, QCAP - LANES))
            return tuple(new_offs)

        def refine(sv, si, dv, di, n, needed, shift, nbits):
            bmask = np.int32((1 << nbits) - 1)
            trips = (n + LANES - 1) // LANES
            clear_hist()

            @pl.loop(0, trips)
            def _(v):
                m = (v * LANES + iota) < n
                key = sv[pl.ds(v * LANES, LANES)] & ABS_MASK
                plsc.addupdate_scatter(
                    hist, [(key >> shift) & bmask], ones16, mask=m)

            bstar, needed2 = scan_hist(needed)

            @pl.loop(0, trips, init_carry=np.int32(0))
            def n2(v, off):
                lanem = (v * LANES + iota) < n
                raw = sv[pl.ds(v * LANES, LANES)]
                idx = si[pl.ds(v * LANES, LANES)]
                b = ((raw & ABS_MASK) >> shift) & bmask
                mgt = lanem & (b > bstar)
                plsc.store_scatter(
                    xrow, [idx], plsc.bitcast(raw, jnp.float32), mask=mgt)
                meq = lanem & (b == bstar)
                plsc.store_compressed(dv.at[pl.ds(off, LANES)], raw, mask=meq)
                plsc.store_compressed(di.at[pl.ds(off, LANES)], idx, mask=meq)
                pc = plsc.all_reduce_population_count(meq)[0]
                return jnp.minimum(off + pc, CAP)

            return n2, needed2

        def refine_seg(sv, si, dv, di, ns, needed, shift, nbits):
            bmask = np.int32((1 << nbits) - 1)
            clear_hist()
            for q in range(NSEG):
                trips = (ns[q] + LANES - 1) // LANES

                @pl.loop(0, trips)
                def _(v):
                    m = (v * LANES + iota) < ns[q]
                    key = sv[pl.ds(q * QCAP + v * LANES, LANES)] & ABS_MASK
                    plsc.addupdate_scatter(
                        hist, [(key >> shift) & bmask], ones16, mask=m)

            bstar, needed2 = scan_hist(needed)

            n2 = np.int32(0)
            for q in range(NSEG):
                trips = (ns[q] + LANES - 1) // LANES

                @pl.loop(0, trips, init_carry=n2)
                def n2(v, off):
                    lanem = (v * LANES + iota) < ns[q]
                    raw = sv[pl.ds(q * QCAP + v * LANES, LANES)]
                    idx = si[pl.ds(q * QCAP + v * LANES, LANES)]
                    b = ((raw & ABS_MASK) >> shift) & bmask
                    mgt = lanem & (b > bstar)
                    plsc.store_scatter(
                        xrow, [idx], plsc.bitcast(raw, jnp.float32), mask=mgt)
                    meq = lanem & (b == bstar)
                    plsc.store_compressed(
                        dv.at[pl.ds(off, LANES)], raw, mask=meq)
                    plsc.store_compressed(
                        di.at[pl.ds(off, LANES)], idx, mask=meq)
                    pc = plsc.all_reduce_population_count(meq)[0]
                    return jnp.minimum(off + pc, CAP)

            return n2, needed2

        needed = np.int32(K)
        nB, needed = refine_seg(av, ai, bv, bi, segn, needed, 23, 8)
        nA2, needed = refine(bv, bi, av, ai, nB, needed, 15, 8)
        nB2, needed = refine(av, ai, bv, bi, nA2, needed, 7, 8)
        nT, needed = refine(bv, bi, av, ai, nB2, needed, 0, 7)

        tie_trips = (jnp.minimum(nT, needed) + LANES - 1) // LANES

        @pl.loop(0, tie_trips)
        def _(v):
            posv = v * LANES + iota
            m = (posv < nT) & (posv < needed)
            raw = av[pl.ds(v * LANES, LANES)]
            idx = ai[pl.ds(v * LANES, LANES)]
            plsc.store_scatter(
                xrow, [idx], plsc.bitcast(raw, jnp.float32), mask=m)

    xbufs = (x0, x1)
    sin = (si0, si1)
    sout = (so0, so1)
    row0 = wid * ROWS_PER_W
    in_d = [pltpu.async_copy(in_hbm.at[row0], x0, si0), None]
    out_d = [None, None]
    for r in range(ROWS_PER_W):
        b = r % 2
        in_d[b].wait()
        if r + 1 < ROWS_PER_W:
            nb = (r + 1) % 2
            if out_d[nb] is not None:
                out_d[nb].wait()
            in_d[nb] = pltpu.async_copy(
                in_hbm.at[row0 + r + 1], xbufs[nb], sin[nb])
        do_row(row0 + r, xbufs[b])
        out_d[b] = pltpu.async_copy(xbufs[b], out_hbm.at[row0 + r], sout[b])
    out_d[(ROWS_PER_W - 2) % 2].wait()
    out_d[(ROWS_PER_W - 1) % 2].wait()


@jax.jit
def kernel(input_):
    mesh = plsc.VectorSubcoreMesh(
        core_axis_name="c", subcore_axis_name="s",
        num_cores=NC, num_subcores=NS)
    f = pl.kernel(
        _body,
        out_type=jax.ShapeDtypeStruct((ROWS, COLS), jnp.float32),
        mesh=mesh,
        scratch_types=[
            pltpu.VMEM((COLS,), jnp.float32),
            pltpu.VMEM((COLS,), jnp.float32),
            pltpu.VMEM((HIST + LANES,), jnp.int32),
            pltpu.VMEM((CAPB,), jnp.int32),
            pltpu.VMEM((CAPB,), jnp.int32),
            pltpu.VMEM((CAPB,), jnp.int32),
            pltpu.VMEM((CAPB,), jnp.int32),
            pltpu.SMEM((LANES,), jnp.int32),
            pltpu.SemaphoreType.DMA,
            pltpu.SemaphoreType.DMA,
            pltpu.SemaphoreType.DMA,
            pltpu.SemaphoreType.DMA,
        ],
        compiler_params=pltpu.CompilerParams(needs_layout_passes=False),
        name="topk_abs_mask_sc",
    )
    return f(input_)


# speedup vs baseline: 1.6378x; 1.0015x over previous
import functools

import jax
import jax.numpy as jnp
import numpy as np
from jax import lax
from jax.experimental import pallas as pl
from jax.experimental.pallas import tpu as pltpu
from jax.experimental.pallas import tpu_sc as plsc

ROWS = 128
COLS = 32768
K = 64
LANES = 16
NV = COLS // LANES
HIST = 256
CAP = 8192
CAPB = CAP + LANES
NSEG = 4
QCAP = CAP // NSEG
NC = 2
NS = 16
NW = NC * NS
ROWS_PER_W = ROWS // NW

ABS_MASK = np.int32(0x7FFFFFFF)


def _body(in_hbm, out_hbm, x0, x1, hist, av, ai, bv, bi, gsum_s,
          si0, si1, so0, so1):
    wid = lax.axis_index("s") * NC + lax.axis_index("c")
    iota = lax.iota(jnp.int32, LANES)
    ones16 = jnp.ones((LANES,), jnp.int32)
    zeros16 = jnp.zeros((LANES,), jnp.int32)

    def clear_hist():
        @pl.loop(0, HIST // LANES)
        def _(h):
            hist[pl.ds(h * LANES, LANES)] = zeros16

    def hist_at(b):
        return hist[pl.ds(b, LANES)][0]

    def scan_hist(needed):
        for g in range(HIST // LANES):
            gsum_s[g] = jnp.sum(hist[pl.ds(g * LANES, LANES)])

        def gcond(st):
            g, cum = st
            return (cum < needed) & (g > 0)

        def gstep(st):
            g, cum = st
            g2 = g - 1
            return g2, cum + gsum_s[g2]

        gstar, gcum = lax.while_loop(
            gcond, gstep, (np.int32(HIST // LANES), np.int32(0)))
        base = gstar * LANES

        def cond(st):
            b, cum = st
            return (cum < needed) & (b > base)

        def step(st):
            b, cum = st
            b2 = b - 1
            return b2, cum + hist_at(b2)

        bstar, cum = lax.while_loop(
            cond, step, (base + LANES, gcum - gsum_s[gstar]))
        return bstar, needed - (cum - hist_at(bstar))

    def do_row(row, xrow):
        @pl.loop(0, NV // NSEG, init_carry=(zeros16,) * NSEG, unroll=4)
        def mxs(v, mxs):
            out = []
            for q in range(NSEG):
                vq = q * (NV // NSEG) + v
                x = xrow[pl.ds(vq * LANES, LANES)]
                key = plsc.bitcast(x, jnp.int32) & ABS_MASK
                out.append(jnp.maximum(mxs[q], key))
            return tuple(out)

        gmin = mxs[0]
        for q in range(1, NSEG):
            gmin = jnp.minimum(gmin, mxs[q])
        tkey = jnp.min(gmin)

        zrow = jnp.zeros((LANES,), jnp.float32)

        @pl.loop(0, NV // NSEG, init_carry=(np.int32(0),) * NSEG, unroll=4)
        def segn(v, offs):
            new_offs = []
            for q in range(NSEG):
                vq = q * (NV // NSEG) + v
                x = xrow[pl.ds(vq * LANES, LANES)]
                raw = plsc.bitcast(x, jnp.int32)
                key = raw & ABS_MASK
                meq = key >= tkey
                off = offs[q]
                base = q * QCAP
                plsc.store_compressed(
                    av.at[pl.ds(base + off, LANES)], raw, mask=meq)
                plsc.store_compressed(
                    ai.at[pl.ds(base + off, LANES)], vq * LANES + iota,
                    mask=meq)
                xrow[pl.ds(vq * LANES, LANES)] = zrow
                pc = plsc.all_reduce_population_count(meq)[0]
                new_offs.append(jnp.minimum(off + pc, QCAP - LANES))
            return tuple(new_offs)

        def refine(sv, si, dv, di, n, needed, shift, nbits):
            bmask = np.int32((1 << nbits) - 1)
            trips = (n + LANES - 1) // LANES
            clear_hist()

            @pl.loop(0, trips)
            def _(v):
                m = (v * LANES + iota) < n
                key = sv[pl.ds(v * LANES, LANES)] & ABS_MASK
                plsc.addupdate_scatter(
                    hist, [(key >> shift) & bmask], ones16, mask=m)

            bstar, needed2 = scan_hist(needed)

            @pl.loop(0, trips, init_carry=np.int32(0))
            def n2(v, off):
                lanem = (v * LANES + iota) < n
                raw = sv[pl.ds(v * LANES, LANES)]
                idx = si[pl.ds(v * LANES, LANES)]
                b = ((raw & ABS_MASK) >> shift) & bmask
                mgt = lanem & (b > bstar)
                plsc.store_scatter(
                    xrow, [idx], plsc.bitcast(raw, jnp.float32), mask=mgt)
                meq = lanem & (b == bstar)
                plsc.store_compressed(dv.at[pl.ds(off, LANES)], raw, mask=meq)
                plsc.store_compressed(di.at[pl.ds(off, LANES)], idx, mask=meq)
                pc = plsc.all_reduce_population_count(meq)[0]
                return jnp.minimum(off + pc, CAP)

            return n2, needed2

        def refine_seg(sv, si, dv, di, ns, needed, shift, nbits):
            bmask = np.int32((1 << nbits) - 1)
            clear_hist()
            for q in range(NSEG):
                trips = (ns[q] + LANES - 1) // LANES

                @pl.loop(0, trips)
                def _(v):
                    m = (v * LANES + iota) < ns[q]
                    key = sv[pl.ds(q * QCAP + v * LANES, LANES)] & ABS_MASK
                    plsc.addupdate_scatter(
                        hist, [(key >> shift) & bmask], ones16, mask=m)

            bstar, needed2 = scan_hist(needed)

            n2 = np.int32(0)
            for q in range(NSEG):
                trips = (ns[q] + LANES - 1) // LANES

                @pl.loop(0, trips, init_carry=n2)
                def n2(v, off):
                    lanem = (v * LANES + iota) < ns[q]
                    raw = sv[pl.ds(q * QCAP + v * LANES, LANES)]
                    idx = si[pl.ds(q * QCAP + v * LANES, LANES)]
                    b = ((raw & ABS_MASK) >> shift) & bmask
                    mgt = lanem & (b > bstar)
                    plsc.store_scatter(
                        xrow, [idx], plsc.bitcast(raw, jnp.float32), mask=mgt)
                    meq = lanem & (b == bstar)
                    plsc.store_compressed(
                        dv.at[pl.ds(off, LANES)], raw, mask=meq)
                    plsc.store_compressed(
                        di.at[pl.ds(off, LANES)], idx, mask=meq)
                    pc = plsc.all_reduce_population_count(meq)[0]
                    return jnp.minimum(off + pc, CAP)

            return n2, needed2

        needed = np.int32(K)
        nB, needed = refine_seg(av, ai, bv, bi, segn, needed, 23, 8)
        nA2, needed = refine(bv, bi, av, ai, nB, needed, 15, 8)
        nB2, needed = refine(av, ai, bv, bi, nA2, needed, 7, 8)
        nT, needed = refine(bv, bi, av, ai, nB2, needed, 0, 7)

        tie_trips = (jnp.minimum(nT, needed) + LANES - 1) // LANES

        @pl.loop(0, tie_trips)
        def _(v):
            posv = v * LANES + iota
            m = (posv < nT) & (posv < needed)
            raw = av[pl.ds(v * LANES, LANES)]
            idx = ai[pl.ds(v * LANES, LANES)]
            plsc.store_scatter(
                xrow, [idx], plsc.bitcast(raw, jnp.float32), mask=m)

    xbufs = (x0, x1)
    sin = (si0, si1)
    sout = (so0, so1)
    row0 = wid * ROWS_PER_W
    in_d = [pltpu.async_copy(in_hbm.at[row0], x0, si0), None]
    out_d = [None, None]
    for r in range(ROWS_PER_W):
        b = r % 2
        in_d[b].wait()
        if r + 1 < ROWS_PER_W:
            nb = (r + 1) % 2
            if out_d[nb] is not None:
                out_d[nb].wait()
            in_d[nb] = pltpu.async_copy(
                in_hbm.at[row0 + r + 1], xbufs[nb], sin[nb])
        do_row(row0 + r, xbufs[b])
        out_d[b] = pltpu.async_copy(xbufs[b], out_hbm.at[row0 + r], sout[b])
    out_d[(ROWS_PER_W - 2) % 2].wait()
    out_d[(ROWS_PER_W - 1) % 2].wait()


@jax.jit
def kernel(input_):
    mesh = plsc.VectorSubcoreMesh(
        core_axis_name="c", subcore_axis_name="s",
        num_cores=NC, num_subcores=NS)
    f = pl.kernel(
        _body,
        out_type=jax.ShapeDtypeStruct((ROWS, COLS), jnp.float32),
        mesh=mesh,
        scratch_types=[
            pltpu.VMEM((COLS,), jnp.float32),
            pltpu.VMEM((COLS,), jnp.float32),
            pltpu.VMEM((HIST + LANES,), jnp.int32),
            pltpu.VMEM((CAPB,), jnp.int32),
            pltpu.VMEM((CAPB,), jnp.int32),
            pltpu.VMEM((CAPB,), jnp.int32),
            pltpu.VMEM((CAPB,), jnp.int32),
            pltpu.SMEM((LANES,), jnp.int32),
            pltpu.SemaphoreType.DMA,
            pltpu.SemaphoreType.DMA,
            pltpu.SemaphoreType.DMA,
            pltpu.SemaphoreType.DMA,
        ],
        compiler_params=pltpu.CompilerParams(needs_layout_passes=False),
        name="topk_abs_mask_sc",
    )
    return f(input_)
